# Initial kernel scaffold; baseline (speedup 1.0000x reference)
import jax
import jax.numpy as jnp
from jax.experimental import pallas as pl


def kernel(x, edge_index, W1, b1, W2, b2):
    raise NotImplementedError("write your pallas kernel here")


# speedup vs baseline: 12.0915x; 12.0915x over previous
import functools

import jax
import jax.numpy as jnp
from jax import lax
from jax.experimental import pallas as pl
from jax.experimental.pallas import tpu as pltpu
from jax.experimental.pallas import tpu_sc as plsc

N_NODES = 10000
N_PAD = 10240
N_EDGES = 320000
DIM = 128

NC = 2
NS = 16
NW = NC * NS
EDGES_PER_W = N_EDGES // NW
CHUNK = 80
NCHUNK = EDGES_PER_W // CHUNK
ROWS_PER_S = N_PAD // NS
DEG_W = 128


@functools.cache
def _get_deg_kernel():
  mesh = plsc.VectorSubcoreMesh(core_axis_name="c", subcore_axis_name="s")

  @functools.partial(
      pl.kernel,
      mesh=mesh,
      out_type=jax.ShapeDtypeStruct((NC, N_PAD, DEG_W), jnp.float32),
      scratch_types=[
          pltpu.VMEM((CHUNK,), jnp.int32),
          pltpu.VMEM((CHUNK, DEG_W), jnp.float32),
          pltpu.VMEM_SHARED((N_PAD, DEG_W), jnp.float32),
      ],
  )
  def deg_kernel(dst_hbm, zeros_hbm, ones_hbm, out_hbm, dst_v, ones_v, deg_sh):
    c = lax.axis_index("c")
    s = lax.axis_index("s")
    w = s * NC + c
    pltpu.sync_copy(ones_hbm, ones_v)
    row0 = s * ROWS_PER_S
    pltpu.sync_copy(zeros_hbm.at[pl.ds(row0, ROWS_PER_S)],
                    deg_sh.at[pl.ds(row0, ROWS_PER_S)])
    plsc.subcore_barrier()
    base = w * EDGES_PER_W

    def body(j, carry):
      off = pl.multiple_of(base + j * CHUNK, 8)
      pltpu.sync_copy(dst_hbm.at[pl.ds(off, CHUNK)], dst_v)
      pltpu.sync_copy(ones_v, deg_sh.at[dst_v], add=True)
      return carry

    lax.fori_loop(0, NCHUNK, body, 0)
    plsc.subcore_barrier()
    pltpu.sync_copy(deg_sh.at[pl.ds(row0, ROWS_PER_S)],
                    out_hbm.at[c, pl.ds(row0, ROWS_PER_S)])

  return deg_kernel


@functools.cache
def _get_agg_kernel():
  mesh = plsc.VectorSubcoreMesh(core_axis_name="c", subcore_axis_name="s")

  @functools.partial(
      pl.kernel,
      mesh=mesh,
      out_type=jax.ShapeDtypeStruct((NC, N_PAD, DIM), jnp.float32),
      scratch_types=[
          pltpu.VMEM((CHUNK,), jnp.int32),
          pltpu.VMEM((CHUNK,), jnp.int32),
          pltpu.VMEM((CHUNK, DIM), jnp.float32),
          pltpu.VMEM_SHARED((N_PAD, DIM), jnp.float32),
          pltpu.SemaphoreType.DMA,
      ],
  )
  def agg_kernel(src_hbm, dst_hbm, h_hbm, zeros_hbm, out_hbm,
                 src_v, dst_v, rows_v, agg_sh, sem):
    c = lax.axis_index("c")
    s = lax.axis_index("s")
    w = s * NC + c
    row0 = s * ROWS_PER_S
    pltpu.sync_copy(zeros_hbm.at[pl.ds(row0, ROWS_PER_S)],
                    agg_sh.at[pl.ds(row0, ROWS_PER_S)])
    plsc.subcore_barrier()
    base = w * EDGES_PER_W

    def body(j, carry):
      off = pl.multiple_of(base + j * CHUNK, 8)
      pltpu.sync_copy(src_hbm.at[pl.ds(off, CHUNK)], src_v)
      pltpu.sync_copy(dst_hbm.at[pl.ds(off, CHUNK)], dst_v)
      pltpu.async_copy(h_hbm.at[src_v], rows_v, sem).wait()
      pltpu.sync_copy(rows_v, agg_sh.at[dst_v], add=True)
      return carry

    lax.fori_loop(0, NCHUNK, body, 0)
    plsc.subcore_barrier()
    pltpu.sync_copy(agg_sh.at[pl.ds(row0, ROWS_PER_S)],
                    out_hbm.at[c, pl.ds(row0, ROWS_PER_S)])

  return agg_kernel


BLK = 1024


def _dis_from_degp(degp_ref):
  deg = (jnp.sum(degp_ref[0], axis=-1) + jnp.sum(degp_ref[1], axis=-1)) * (
      1.0 / DEG_W) + 1.0
  return lax.rsqrt(deg)


def _tc_a_body(x_ref, w_ref, degp_ref, out_ref):
  dis = _dis_from_degp(degp_ref)
  h = jnp.dot(x_ref[...], w_ref[...], preferred_element_type=jnp.float32,
              precision=lax.Precision.HIGHEST)
  out_ref[...] = h * dis[:, None]


_tc_a = pl.pallas_call(
    _tc_a_body,
    grid=(N_PAD // BLK,),
    in_specs=[
        pl.BlockSpec((BLK, DIM), lambda i: (i, 0)),
        pl.BlockSpec((DIM, DIM), lambda i: (0, 0)),
        pl.BlockSpec((NC, BLK, DEG_W), lambda i: (0, i, 0)),
    ],
    out_specs=pl.BlockSpec((BLK, DIM), lambda i: (i, 0)),
    out_shape=jax.ShapeDtypeStruct((N_PAD, DIM), jnp.float32),
)


def _tc_b_body(aggp_ref, h1_ref, degp_ref, b1_ref, w2_ref, out_ref):
  dis = _dis_from_degp(degp_ref)
  t = (aggp_ref[0] + aggp_ref[1] + h1_ref[...]) * dis[:, None] + b1_ref[...]
  z = jnp.maximum(t, 0.0)
  h2 = jnp.dot(z, w2_ref[...], preferred_element_type=jnp.float32,
               precision=lax.Precision.HIGHEST)
  out_ref[...] = h2 * dis[:, None]


_tc_b = pl.pallas_call(
    _tc_b_body,
    grid=(N_PAD // BLK,),
    in_specs=[
        pl.BlockSpec((NC, BLK, DIM), lambda i: (0, i, 0)),
        pl.BlockSpec((BLK, DIM), lambda i: (i, 0)),
        pl.BlockSpec((NC, BLK, DEG_W), lambda i: (0, i, 0)),
        pl.BlockSpec((1, DIM), lambda i: (0, 0)),
        pl.BlockSpec((DIM, DIM), lambda i: (0, 0)),
    ],
    out_specs=pl.BlockSpec((BLK, DIM), lambda i: (i, 0)),
    out_shape=jax.ShapeDtypeStruct((N_PAD, DIM), jnp.float32),
)


def _tc_c_body(aggp_ref, h2_ref, degp_ref, b2_ref, out_ref):
  dis = _dis_from_degp(degp_ref)
  out_ref[...] = (aggp_ref[0] + aggp_ref[1] + h2_ref[...]) * dis[:, None] + \
      b2_ref[...]


_tc_c = pl.pallas_call(
    _tc_c_body,
    grid=(N_PAD // BLK,),
    in_specs=[
        pl.BlockSpec((NC, BLK, DIM), lambda i: (0, i, 0)),
        pl.BlockSpec((BLK, DIM), lambda i: (i, 0)),
        pl.BlockSpec((NC, BLK, DEG_W), lambda i: (0, i, 0)),
        pl.BlockSpec((1, DIM), lambda i: (0, 0)),
    ],
    out_specs=pl.BlockSpec((BLK, DIM), lambda i: (i, 0)),
    out_shape=jax.ShapeDtypeStruct((N_PAD, DIM), jnp.float32),
)


def kernel(x, edge_index, W1, b1, W2, b2):
  src = edge_index[0].astype(jnp.int32)
  dst = edge_index[1].astype(jnp.int32)
  xp = jnp.pad(x, ((0, N_PAD - N_NODES), (0, 0)))
  zeros_deg = jnp.zeros((N_PAD, DEG_W), jnp.float32)
  zeros_feat = jnp.zeros((N_PAD, DIM), jnp.float32)
  degp = _get_deg_kernel()(dst, zeros_deg, jnp.ones((CHUNK, DEG_W), jnp.float32))
  h1 = _tc_a(xp, W1, degp)
  aggp1 = _get_agg_kernel()(src, dst, h1, zeros_feat)
  h2 = _tc_b(aggp1, h1, degp, b1.reshape(1, DIM), W2)
  aggp2 = _get_agg_kernel()(src, dst, h2, zeros_feat)
  return _tc_c(aggp2, h2, degp, b2.reshape(1, DIM))[:N_NODES]


# speedup vs baseline: 23.5067x; 1.9441x over previous
import functools

import jax
import jax.numpy as jnp
from jax import lax
from jax.experimental import pallas as pl
from jax.experimental.pallas import tpu as pltpu
from jax.experimental.pallas import tpu_sc as plsc

N_NODES = 10000
N_PAD = 10240
N_EDGES = 320000
DIM = 128

NC = 2
NS = 16
NW = NC * NS
EDGES_PER_W = N_EDGES // NW
CHUNK = 80
NCHUNK = EDGES_PER_W // CHUNK
ROWS_PER_S = N_PAD // NS
DEG_W = 128


@functools.cache
def _get_deg_kernel(width=DEG_W):
  mesh = plsc.VectorSubcoreMesh(core_axis_name="c", subcore_axis_name="s")

  @functools.partial(
      pl.kernel,
      mesh=mesh,
      out_type=jax.ShapeDtypeStruct((NC, N_PAD, width), jnp.float32),
      scratch_types=[
          pltpu.VMEM((NCHUNK, CHUNK), jnp.int32),
          pltpu.VMEM((CHUNK, width), jnp.float32),
          pltpu.VMEM_SHARED((N_PAD, width), jnp.float32),
          pltpu.SemaphoreType.DMA,
      ],
  )
  def deg_kernel(dsts_hbm, zeros_hbm, ones_hbm, out_hbm,
                 didx_v, ones_v, deg_sh, ssem):
    c = lax.axis_index("c")
    s = lax.axis_index("s")
    w = s * NC + c
    pltpu.sync_copy(dsts_hbm.at[w], didx_v)
    pltpu.sync_copy(ones_hbm, ones_v)
    row0 = s * ROWS_PER_S
    pltpu.sync_copy(zeros_hbm.at[pl.ds(row0, ROWS_PER_S)],
                    deg_sh.at[pl.ds(row0, ROWS_PER_S)])
    plsc.subcore_barrier()

    DEPTH = 4

    def body(j, carry):
      pltpu.async_copy(ones_v, deg_sh.at[didx_v.at[j]], ssem, add=True)

      @pl.when(j >= DEPTH)
      def _():
        pltpu.make_async_copy(ones_v, deg_sh.at[didx_v.at[0]], ssem).wait()

      return carry

    lax.fori_loop(0, NCHUNK, body, 0)
    for _ in range(DEPTH):
      pltpu.make_async_copy(ones_v, deg_sh.at[didx_v.at[0]], ssem).wait()
    plsc.subcore_barrier()
    pltpu.sync_copy(deg_sh.at[pl.ds(row0, ROWS_PER_S)],
                    out_hbm.at[c, pl.ds(row0, ROWS_PER_S)])

  return deg_kernel


@functools.cache
def _get_agg_kernel():
  mesh = plsc.VectorSubcoreMesh(core_axis_name="c", subcore_axis_name="s")
  NSLOT = 4

  @functools.partial(
      pl.kernel,
      mesh=mesh,
      out_type=jax.ShapeDtypeStruct((NC, N_PAD, DIM), jnp.float32),
      scratch_types=(
          [pltpu.VMEM((CHUNK,), jnp.int32)] * NSLOT
          + [pltpu.VMEM((CHUNK,), jnp.int32)] * NSLOT
          + [pltpu.VMEM((CHUNK, DIM), jnp.float32)] * NSLOT
          + [pltpu.VMEM_SHARED((N_PAD, DIM), jnp.float32)]
          + [pltpu.SemaphoreType.DMA] * (3 * NSLOT)
      ),
  )
  def agg_kernel(src_hbm, dst_hbm, h_hbm, zeros_hbm, out_hbm, *scr):
    sidx = scr[0:NSLOT]
    didx = scr[NSLOT:2 * NSLOT]
    rows = scr[2 * NSLOT:3 * NSLOT]
    agg_sh = scr[3 * NSLOT]
    isem = scr[3 * NSLOT + 1:3 * NSLOT + 1 + NSLOT]
    gsem = scr[3 * NSLOT + 1 + NSLOT:3 * NSLOT + 1 + 2 * NSLOT]
    ssem = scr[3 * NSLOT + 1 + 2 * NSLOT:3 * NSLOT + 1 + 3 * NSLOT]
    c = lax.axis_index("c")
    s = lax.axis_index("s")
    w = s * NC + c
    row0 = s * ROWS_PER_S
    base = w * EDGES_PER_W

    def iload(j, b):
      off = pl.multiple_of(base + j * CHUNK, 8)
      pltpu.make_async_copy(src_hbm.at[pl.ds(off, CHUNK)], sidx[b],
                            isem[b]).start()
      pltpu.make_async_copy(dst_hbm.at[pl.ds(off, CHUNK)], didx[b],
                            isem[b]).start()

    def iwait(b):
      pltpu.make_async_copy(src_hbm.at[pl.ds(0, CHUNK)], sidx[b],
                            isem[b]).wait()
      pltpu.make_async_copy(dst_hbm.at[pl.ds(0, CHUNK)], didx[b],
                            isem[b]).wait()

    def gstart(b):
      pltpu.make_async_copy(h_hbm.at[sidx[b]], rows[b], gsem[b]).start()

    def gwait(b):
      pltpu.make_async_copy(h_hbm.at[sidx[b]], rows[b], gsem[b]).wait()

    def sstart(b):
      pltpu.async_copy(rows[b], agg_sh.at[didx[b]], ssem[b], add=True)

    def swait(b):
      pltpu.make_async_copy(rows[b], agg_sh.at[didx[b]], ssem[b]).wait()

    for b in range(NSLOT):
      iload(b, b)
    for b in range(NSLOT):
      iwait(b)
      gstart(b)
    pltpu.sync_copy(zeros_hbm.at[pl.ds(row0, ROWS_PER_S)],
                    agg_sh.at[pl.ds(row0, ROWS_PER_S)])
    plsc.subcore_barrier()

    def body(i, carry):
      for b in range(NSLOT):
        j = NSLOT * i + b

        @pl.when(j < NCHUNK)
        def _():
          gwait(b)
          sstart(b)

        @pl.when(j + NSLOT < NCHUNK)
        def _():
          swait(b)
          iload(j + NSLOT, b)
          iwait(b)
          gstart(b)

      return carry

    lax.fori_loop(0, (NCHUNK + NSLOT - 1) // NSLOT, body, 0)
    for b in range(NSLOT):
      swait(b)
    plsc.subcore_barrier()
    pltpu.sync_copy(agg_sh.at[pl.ds(row0, ROWS_PER_S)],
                    out_hbm.at[c, pl.ds(row0, ROWS_PER_S)])

  return agg_kernel


BLK = 1024


def _dis_from_degp(degp_ref):
  deg = (jnp.sum(degp_ref[0], axis=-1) + jnp.sum(degp_ref[1], axis=-1)) * (
      1.0 / DEG_W) + 1.0
  return lax.rsqrt(deg)


def _tc_a_body(x_ref, w_ref, degp_ref, out_ref):
  dis = _dis_from_degp(degp_ref)
  h = jnp.dot(x_ref[...], w_ref[...], preferred_element_type=jnp.float32,
              precision=lax.Precision.HIGHEST)
  out_ref[...] = h * dis[:, None]


_tc_a = pl.pallas_call(
    _tc_a_body,
    grid=(N_PAD // BLK,),
    in_specs=[
        pl.BlockSpec((BLK, DIM), lambda i: (i, 0)),
        pl.BlockSpec((DIM, DIM), lambda i: (0, 0)),
        pl.BlockSpec((NC, BLK, DEG_W), lambda i: (0, i, 0)),
    ],
    out_specs=pl.BlockSpec((BLK, DIM), lambda i: (i, 0)),
    out_shape=jax.ShapeDtypeStruct((N_PAD, DIM), jnp.float32),
)


def _tc_b_body(aggp_ref, h1_ref, degp_ref, b1_ref, w2_ref, out_ref):
  dis = _dis_from_degp(degp_ref)
  t = (aggp_ref[0] + aggp_ref[1] + h1_ref[...]) * dis[:, None] + b1_ref[...]
  z = jnp.maximum(t, 0.0)
  h2 = jnp.dot(z, w2_ref[...], preferred_element_type=jnp.float32,
               precision=lax.Precision.HIGHEST)
  out_ref[...] = h2 * dis[:, None]


_tc_b = pl.pallas_call(
    _tc_b_body,
    grid=(N_PAD // BLK,),
    in_specs=[
        pl.BlockSpec((NC, BLK, DIM), lambda i: (0, i, 0)),
        pl.BlockSpec((BLK, DIM), lambda i: (i, 0)),
        pl.BlockSpec((NC, BLK, DEG_W), lambda i: (0, i, 0)),
        pl.BlockSpec((1, DIM), lambda i: (0, 0)),
        pl.BlockSpec((DIM, DIM), lambda i: (0, 0)),
    ],
    out_specs=pl.BlockSpec((BLK, DIM), lambda i: (i, 0)),
    out_shape=jax.ShapeDtypeStruct((N_PAD, DIM), jnp.float32),
)


def _tc_c_body(aggp_ref, h2_ref, degp_ref, b2_ref, out_ref):
  dis = _dis_from_degp(degp_ref)
  out_ref[...] = (aggp_ref[0] + aggp_ref[1] + h2_ref[...]) * dis[:, None] + \
      b2_ref[...]


_tc_c = pl.pallas_call(
    _tc_c_body,
    grid=(N_PAD // BLK,),
    in_specs=[
        pl.BlockSpec((NC, BLK, DIM), lambda i: (0, i, 0)),
        pl.BlockSpec((BLK, DIM), lambda i: (i, 0)),
        pl.BlockSpec((NC, BLK, DEG_W), lambda i: (0, i, 0)),
        pl.BlockSpec((1, DIM), lambda i: (0, 0)),
    ],
    out_specs=pl.BlockSpec((BLK, DIM), lambda i: (i, 0)),
    out_shape=jax.ShapeDtypeStruct((N_PAD, DIM), jnp.float32),
)


def kernel(x, edge_index, W1, b1, W2, b2):
  srcs = edge_index[0].astype(jnp.int32).reshape(NW, NCHUNK, CHUNK)
  dsts = edge_index[1].astype(jnp.int32).reshape(NW, NCHUNK, CHUNK)
  xp = jnp.pad(x, ((0, N_PAD - N_NODES), (0, 0)))
  zeros_deg = jnp.zeros((N_PAD, DEG_W), jnp.float32)
  zeros_feat = jnp.zeros((N_PAD, DIM), jnp.float32)
  src1d = edge_index[0].astype(jnp.int32)
  dst1d = edge_index[1].astype(jnp.int32)
  degp = _get_deg_kernel()(dsts, zeros_deg,
                           jnp.ones((CHUNK, DEG_W), jnp.float32))
  h1 = _tc_a(xp, W1, degp)
  aggp1 = _get_agg_kernel()(src1d, dst1d, h1, zeros_feat)
  h2 = _tc_b(aggp1, h1, degp, b1.reshape(1, DIM), W2)
  aggp2 = _get_agg_kernel()(src1d, dst1d, h2, zeros_feat)
  return _tc_c(aggp2, h2, degp, b2.reshape(1, DIM))[:N_NODES]


# speedup vs baseline: 25.9691x; 1.1048x over previous
import functools

import jax
import jax.numpy as jnp
from jax import lax
from jax.experimental import pallas as pl
from jax.experimental.pallas import tpu as pltpu
from jax.experimental.pallas import tpu_sc as plsc

N_NODES = 10000
N_PAD = 10112
N_EDGES = 320000
DIM = 128

NC = 2
NS = 16
NW = NC * NS
EDGES_PER_W = N_EDGES // NW
CHUNK = 80
NCHUNK = EDGES_PER_W // CHUNK
ROWS_PER_S = N_PAD // NS
DEG_W = 128


@functools.cache
def _get_deg_kernel(width=DEG_W):
  mesh = plsc.VectorSubcoreMesh(core_axis_name="c", subcore_axis_name="s")

  @functools.partial(
      pl.kernel,
      mesh=mesh,
      out_type=jax.ShapeDtypeStruct((NC, N_PAD, width), jnp.float32),
      scratch_types=[
          pltpu.VMEM((NCHUNK, CHUNK), jnp.int32),
          pltpu.VMEM((CHUNK, width), jnp.float32),
          pltpu.VMEM_SHARED((N_PAD, width), jnp.float32),
          pltpu.SemaphoreType.DMA,
      ],
  )
  def deg_kernel(dsts_hbm, zeros_hbm, ones_hbm, out_hbm,
                 didx_v, ones_v, deg_sh, ssem):
    c = lax.axis_index("c")
    s = lax.axis_index("s")
    w = s * NC + c
    pltpu.sync_copy(dsts_hbm.at[w], didx_v)
    pltpu.sync_copy(ones_hbm, ones_v)
    row0 = s * ROWS_PER_S
    pltpu.sync_copy(zeros_hbm.at[pl.ds(row0, ROWS_PER_S)],
                    deg_sh.at[pl.ds(row0, ROWS_PER_S)])
    plsc.subcore_barrier()

    DEPTH = 4

    def body(j, carry):
      pltpu.async_copy(ones_v, deg_sh.at[didx_v.at[j]], ssem, add=True)

      @pl.when(j >= DEPTH)
      def _():
        pltpu.make_async_copy(ones_v, deg_sh.at[didx_v.at[0]], ssem).wait()

      return carry

    lax.fori_loop(0, NCHUNK, body, 0)
    for _ in range(DEPTH):
      pltpu.make_async_copy(ones_v, deg_sh.at[didx_v.at[0]], ssem).wait()
    plsc.subcore_barrier()
    pltpu.sync_copy(deg_sh.at[pl.ds(row0, ROWS_PER_S)],
                    out_hbm.at[c, pl.ds(row0, ROWS_PER_S)])

  return deg_kernel


@functools.cache
def _get_agg_kernel():
  mesh = plsc.VectorSubcoreMesh(core_axis_name="c", subcore_axis_name="s")
  NSLOT = 3
  ECH = 128
  NCH_TOT = N_EDGES // ECH
  MAX_M = (NCH_TOT + NW - 1) // NW

  @functools.partial(
      pl.kernel,
      mesh=mesh,
      out_type=jax.ShapeDtypeStruct((NC, N_PAD, DIM), jnp.float32),
      scratch_types=(
          [pltpu.VMEM((ECH,), jnp.int32)] * NSLOT
          + [pltpu.VMEM((ECH,), jnp.int32)] * NSLOT
          + [pltpu.VMEM((ECH, DIM), jnp.float32)] * NSLOT
          + [pltpu.VMEM_SHARED((N_PAD, DIM), jnp.float32)]
          + [pltpu.SemaphoreType.DMA] * (3 * NSLOT)
      ),
  )
  def agg_kernel(src_hbm, dst_hbm, h_hbm, zeros_hbm, out_hbm, *scr):
    sidx = scr[0:NSLOT]
    didx = scr[NSLOT:2 * NSLOT]
    rows = scr[2 * NSLOT:3 * NSLOT]
    agg_sh = scr[3 * NSLOT]
    isem = scr[3 * NSLOT + 1:3 * NSLOT + 1 + NSLOT]
    gsem = scr[3 * NSLOT + 1 + NSLOT:3 * NSLOT + 1 + 2 * NSLOT]
    ssem = scr[3 * NSLOT + 1 + 2 * NSLOT:3 * NSLOT + 1 + 3 * NSLOT]
    c = lax.axis_index("c")
    s = lax.axis_index("s")
    w = s * NC + c
    row0 = s * ROWS_PER_S

    def iload(k, b):
      off = pl.multiple_of(k * ECH, 8)
      pltpu.make_async_copy(src_hbm.at[pl.ds(off, ECH)], sidx[b],
                            isem[b]).start()
      pltpu.make_async_copy(dst_hbm.at[pl.ds(off, ECH)], didx[b],
                            isem[b]).start()

    def iwait(b):
      pltpu.make_async_copy(src_hbm.at[pl.ds(0, ECH)], sidx[b],
                            isem[b]).wait()
      pltpu.make_async_copy(dst_hbm.at[pl.ds(0, ECH)], didx[b],
                            isem[b]).wait()

    def gstart(b):
      pltpu.make_async_copy(h_hbm.at[sidx[b]], rows[b], gsem[b]).start()

    def gwait(b):
      pltpu.make_async_copy(h_hbm.at[sidx[b]], rows[b], gsem[b]).wait()

    def sstart(b):
      pltpu.async_copy(rows[b], agg_sh.at[didx[b]], ssem[b], add=True)

    def swait(b):
      pltpu.make_async_copy(rows[b], agg_sh.at[didx[b]], ssem[b]).wait()

    for b in range(NSLOT):
      iload(w + NW * b, b)
    for b in range(NSLOT):
      iwait(b)
      gstart(b)
    pltpu.sync_copy(zeros_hbm.at[pl.ds(row0, ROWS_PER_S)],
                    agg_sh.at[pl.ds(row0, ROWS_PER_S)])
    plsc.subcore_barrier()

    def body(i, carry):
      for b in range(NSLOT):
        m = NSLOT * i + b
        k = w + NW * m

        @pl.when(k < NCH_TOT)
        def _():
          gwait(b)
          sstart(b)

        @pl.when(w + NW * (m + NSLOT) < NCH_TOT)
        def _():
          swait(b)
          iload(w + NW * (m + NSLOT), b)
          iwait(b)
          gstart(b)

      return carry

    lax.fori_loop(0, (MAX_M + NSLOT - 1) // NSLOT, body, 0)
    for b in range(NSLOT):
      swait(b)
    plsc.subcore_barrier()
    pltpu.sync_copy(agg_sh.at[pl.ds(row0, ROWS_PER_S)],
                    out_hbm.at[c, pl.ds(row0, ROWS_PER_S)])

  return agg_kernel


BLK = 1264


def _dis_from_degp(degp_ref):
  deg = (jnp.sum(degp_ref[0], axis=-1) + jnp.sum(degp_ref[1], axis=-1)) * (
      1.0 / DEG_W) + 1.0
  return lax.rsqrt(deg)


def _tc_a_body(x_ref, w_ref, degp_ref, out_ref):
  dis = _dis_from_degp(degp_ref)
  h = jnp.dot(x_ref[...], w_ref[...], preferred_element_type=jnp.float32,
              precision=lax.Precision.HIGHEST)
  out_ref[...] = h * dis[:, None]


_tc_a = pl.pallas_call(
    _tc_a_body,
    grid=(N_PAD // BLK,),
    in_specs=[
        pl.BlockSpec((BLK, DIM), lambda i: (i, 0)),
        pl.BlockSpec((DIM, DIM), lambda i: (0, 0)),
        pl.BlockSpec((NC, BLK, DEG_W), lambda i: (0, i, 0)),
    ],
    out_specs=pl.BlockSpec((BLK, DIM), lambda i: (i, 0)),
    out_shape=jax.ShapeDtypeStruct((N_PAD, DIM), jnp.float32),
)


def _tc_b_body(aggp_ref, h1_ref, degp_ref, b1_ref, w2_ref, out_ref):
  dis = _dis_from_degp(degp_ref)
  t = (aggp_ref[0] + aggp_ref[1] + h1_ref[...]) * dis[:, None] + b1_ref[...]
  z = jnp.maximum(t, 0.0)
  h2 = jnp.dot(z, w2_ref[...], preferred_element_type=jnp.float32,
               precision=lax.Precision.HIGHEST)
  out_ref[...] = h2 * dis[:, None]


_tc_b = pl.pallas_call(
    _tc_b_body,
    grid=(N_PAD // BLK,),
    in_specs=[
        pl.BlockSpec((NC, BLK, DIM), lambda i: (0, i, 0)),
        pl.BlockSpec((BLK, DIM), lambda i: (i, 0)),
        pl.BlockSpec((NC, BLK, DEG_W), lambda i: (0, i, 0)),
        pl.BlockSpec((1, DIM), lambda i: (0, 0)),
        pl.BlockSpec((DIM, DIM), lambda i: (0, 0)),
    ],
    out_specs=pl.BlockSpec((BLK, DIM), lambda i: (i, 0)),
    out_shape=jax.ShapeDtypeStruct((N_PAD, DIM), jnp.float32),
)


def _tc_c_body(aggp_ref, h2_ref, degp_ref, b2_ref, out_ref):
  dis = _dis_from_degp(degp_ref)
  out_ref[...] = (aggp_ref[0] + aggp_ref[1] + h2_ref[...]) * dis[:, None] + \
      b2_ref[...]


_tc_c = pl.pallas_call(
    _tc_c_body,
    grid=(N_PAD // BLK,),
    in_specs=[
        pl.BlockSpec((NC, BLK, DIM), lambda i: (0, i, 0)),
        pl.BlockSpec((BLK, DIM), lambda i: (i, 0)),
        pl.BlockSpec((NC, BLK, DEG_W), lambda i: (0, i, 0)),
        pl.BlockSpec((1, DIM), lambda i: (0, 0)),
    ],
    out_specs=pl.BlockSpec((BLK, DIM), lambda i: (i, 0)),
    out_shape=jax.ShapeDtypeStruct((N_PAD, DIM), jnp.float32),
)


def kernel(x, edge_index, W1, b1, W2, b2):
  srcs = edge_index[0].astype(jnp.int32).reshape(NW, NCHUNK, CHUNK)
  dsts = edge_index[1].astype(jnp.int32).reshape(NW, NCHUNK, CHUNK)
  xp = jnp.pad(x, ((0, N_PAD - N_NODES), (0, 0)))
  zeros_deg = jnp.zeros((N_PAD, DEG_W), jnp.float32)
  zeros_feat = jnp.zeros((N_PAD, DIM), jnp.float32)
  src1d = edge_index[0].astype(jnp.int32)
  dst1d = edge_index[1].astype(jnp.int32)
  degp = _get_deg_kernel()(dsts, zeros_deg,
                           jnp.ones((CHUNK, DEG_W), jnp.float32))
  h1 = _tc_a(xp, W1, degp)
  aggp1 = _get_agg_kernel()(src1d, dst1d, h1, zeros_feat)
  h2 = _tc_b(aggp1, h1, degp, b1.reshape(1, DIM), W2)
  aggp2 = _get_agg_kernel()(src1d, dst1d, h2, zeros_feat)
  return _tc_c(aggp2, h2, degp, b2.reshape(1, DIM))[:N_NODES]


# speedup vs baseline: 26.0120x; 1.0016x over previous
import functools

import jax
import jax.numpy as jnp
from jax import lax
from jax.experimental import pallas as pl
from jax.experimental.pallas import tpu as pltpu
from jax.experimental.pallas import tpu_sc as plsc

N_NODES = 10000
N_PAD = 10112
N_EDGES = 320000
DIM = 128

NC = 2
NS = 16
NW = NC * NS
EDGES_PER_W = N_EDGES // NW
CHUNK = 80
NCHUNK = EDGES_PER_W // CHUNK
ROWS_PER_S = N_PAD // NS
DEG_W = 128


@functools.cache
def _get_deg_kernel():
  mesh = plsc.VectorSubcoreMesh(core_axis_name="c", subcore_axis_name="s")
  NSLOT = 3
  ECH = 128
  NCH_TOT = N_EDGES // ECH
  MAX_M = (NCH_TOT + NW - 1) // NW

  @functools.partial(
      pl.kernel,
      mesh=mesh,
      out_type=jax.ShapeDtypeStruct((NC, N_PAD, DEG_W), jnp.float32),
      scratch_types=(
          [pltpu.VMEM((ECH,), jnp.int32)] * NSLOT
          + [pltpu.VMEM((ECH, DEG_W), jnp.float32)]
          + [pltpu.VMEM_SHARED((N_PAD, DEG_W), jnp.float32)]
          + [pltpu.SemaphoreType.DMA] * (2 * NSLOT)
      ),
  )
  def deg_kernel(dst_hbm, zeros_hbm, ones_hbm, out_hbm, *scr):
    didx = scr[0:NSLOT]
    ones_v = scr[NSLOT]
    deg_sh = scr[NSLOT + 1]
    isem = scr[NSLOT + 2:NSLOT + 2 + NSLOT]
    ssem = scr[NSLOT + 2 + NSLOT:NSLOT + 2 + 2 * NSLOT]
    c = lax.axis_index("c")
    s = lax.axis_index("s")
    w = s * NC + c
    row0 = s * ROWS_PER_S

    def iload(k, b):
      off = pl.multiple_of(k * ECH, 8)
      pltpu.make_async_copy(dst_hbm.at[pl.ds(off, ECH)], didx[b],
                            isem[b]).start()

    def iwait(b):
      pltpu.make_async_copy(dst_hbm.at[pl.ds(0, ECH)], didx[b],
                            isem[b]).wait()

    def sstart(b):
      pltpu.async_copy(ones_v, deg_sh.at[didx[b]], ssem[b], add=True)

    def swait(b):
      pltpu.make_async_copy(ones_v, deg_sh.at[didx[b]], ssem[b]).wait()

    for b in range(NSLOT):
      iload(w + NW * b, b)
    pltpu.sync_copy(ones_hbm, ones_v)
    pltpu.sync_copy(zeros_hbm.at[pl.ds(row0, ROWS_PER_S)],
                    deg_sh.at[pl.ds(row0, ROWS_PER_S)])
    plsc.subcore_barrier()

    def body(i, carry):
      for b in range(NSLOT):
        m = NSLOT * i + b
        k = w + NW * m

        @pl.when(k < NCH_TOT)
        def _():
          iwait(b)
          sstart(b)

        @pl.when(w + NW * (m + NSLOT) < NCH_TOT)
        def _():
          swait(b)
          iload(w + NW * (m + NSLOT), b)

      return carry

    lax.fori_loop(0, (MAX_M + NSLOT - 1) // NSLOT, body, 0)
    for b in range(NSLOT):
      swait(b)
    plsc.subcore_barrier()
    pltpu.sync_copy(deg_sh.at[pl.ds(row0, ROWS_PER_S)],
                    out_hbm.at[c, pl.ds(row0, ROWS_PER_S)])

  return deg_kernel


@functools.cache
def _get_agg_kernel():
  mesh = plsc.VectorSubcoreMesh(core_axis_name="c", subcore_axis_name="s")
  NSLOT = 3
  ECH = 128
  NCH_TOT = N_EDGES // ECH
  MAX_M = (NCH_TOT + NW - 1) // NW

  @functools.partial(
      pl.kernel,
      mesh=mesh,
      out_type=jax.ShapeDtypeStruct((NC, N_PAD, DIM), jnp.float32),
      scratch_types=(
          [pltpu.VMEM((ECH,), jnp.int32)] * NSLOT
          + [pltpu.VMEM((ECH,), jnp.int32)] * NSLOT
          + [pltpu.VMEM((ECH, DIM), jnp.float32)] * NSLOT
          + [pltpu.VMEM_SHARED((N_PAD, DIM), jnp.float32)]
          + [pltpu.SemaphoreType.DMA] * (3 * NSLOT)
      ),
  )
  def agg_kernel(src_hbm, dst_hbm, h_hbm, zeros_hbm, out_hbm, *scr):
    sidx = scr[0:NSLOT]
    didx = scr[NSLOT:2 * NSLOT]
    rows = scr[2 * NSLOT:3 * NSLOT]
    agg_sh = scr[3 * NSLOT]
    isem = scr[3 * NSLOT + 1:3 * NSLOT + 1 + NSLOT]
    gsem = scr[3 * NSLOT + 1 + NSLOT:3 * NSLOT + 1 + 2 * NSLOT]
    ssem = scr[3 * NSLOT + 1 + 2 * NSLOT:3 * NSLOT + 1 + 3 * NSLOT]
    c = lax.axis_index("c")
    s = lax.axis_index("s")
    w = s * NC + c
    row0 = s * ROWS_PER_S

    def iload(k, b):
      off = pl.multiple_of(k * ECH, 8)
      pltpu.make_async_copy(src_hbm.at[pl.ds(off, ECH)], sidx[b],
                            isem[b]).start()
      pltpu.make_async_copy(dst_hbm.at[pl.ds(off, ECH)], didx[b],
                            isem[b]).start()

    def iwait(b):
      pltpu.make_async_copy(src_hbm.at[pl.ds(0, ECH)], sidx[b],
                            isem[b]).wait()
      pltpu.make_async_copy(dst_hbm.at[pl.ds(0, ECH)], didx[b],
                            isem[b]).wait()

    def gstart(b):
      pltpu.make_async_copy(h_hbm.at[sidx[b]], rows[b], gsem[b]).start()

    def gwait(b):
      pltpu.make_async_copy(h_hbm.at[sidx[b]], rows[b], gsem[b]).wait()

    def sstart(b):
      pltpu.async_copy(rows[b], agg_sh.at[didx[b]], ssem[b], add=True)

    def swait(b):
      pltpu.make_async_copy(rows[b], agg_sh.at[didx[b]], ssem[b]).wait()

    for b in range(NSLOT):
      iload(w + NW * b, b)
    for b in range(NSLOT):
      iwait(b)
      gstart(b)
    pltpu.sync_copy(zeros_hbm.at[pl.ds(row0, ROWS_PER_S)],
                    agg_sh.at[pl.ds(row0, ROWS_PER_S)])
    plsc.subcore_barrier()

    def body(i, carry):
      for b in range(NSLOT):
        m = NSLOT * i + b
        k = w + NW * m

        @pl.when(k < NCH_TOT)
        def _():
          gwait(b)
          sstart(b)

        @pl.when(w + NW * (m + NSLOT) < NCH_TOT)
        def _():
          swait(b)
          iload(w + NW * (m + NSLOT), b)
          iwait(b)
          gstart(b)

      return carry

    lax.fori_loop(0, (MAX_M + NSLOT - 1) // NSLOT, body, 0)
    for b in range(NSLOT):
      swait(b)
    plsc.subcore_barrier()
    pltpu.sync_copy(agg_sh.at[pl.ds(row0, ROWS_PER_S)],
                    out_hbm.at[c, pl.ds(row0, ROWS_PER_S)])

  return agg_kernel


BLK = 1264


def _dis_from_degp(degp_ref):
  deg = (jnp.sum(degp_ref[0], axis=-1) + jnp.sum(degp_ref[1], axis=-1)) * (
      1.0 / DEG_W) + 1.0
  return lax.rsqrt(deg)


def _tc_a_body(x_ref, w_ref, degp_ref, out_ref):
  dis = _dis_from_degp(degp_ref)
  h = jnp.dot(x_ref[...], w_ref[...], preferred_element_type=jnp.float32,
              precision=lax.Precision.HIGHEST)
  out_ref[...] = h * dis[:, None]


_tc_a = pl.pallas_call(
    _tc_a_body,
    grid=(N_PAD // BLK,),
    in_specs=[
        pl.BlockSpec((BLK, DIM), lambda i: (i, 0)),
        pl.BlockSpec((DIM, DIM), lambda i: (0, 0)),
        pl.BlockSpec((NC, BLK, DEG_W), lambda i: (0, i, 0)),
    ],
    out_specs=pl.BlockSpec((BLK, DIM), lambda i: (i, 0)),
    out_shape=jax.ShapeDtypeStruct((N_PAD, DIM), jnp.float32),
)


def _tc_b_body(aggp_ref, h1_ref, degp_ref, b1_ref, w2_ref, out_ref):
  dis = _dis_from_degp(degp_ref)
  t = (aggp_ref[0] + aggp_ref[1] + h1_ref[...]) * dis[:, None] + b1_ref[...]
  z = jnp.maximum(t, 0.0)
  h2 = jnp.dot(z, w2_ref[...], preferred_element_type=jnp.float32,
               precision=lax.Precision.HIGHEST)
  out_ref[...] = h2 * dis[:, None]


_tc_b = pl.pallas_call(
    _tc_b_body,
    grid=(N_PAD // BLK,),
    in_specs=[
        pl.BlockSpec((NC, BLK, DIM), lambda i: (0, i, 0)),
        pl.BlockSpec((BLK, DIM), lambda i: (i, 0)),
        pl.BlockSpec((NC, BLK, DEG_W), lambda i: (0, i, 0)),
        pl.BlockSpec((1, DIM), lambda i: (0, 0)),
        pl.BlockSpec((DIM, DIM), lambda i: (0, 0)),
    ],
    out_specs=pl.BlockSpec((BLK, DIM), lambda i: (i, 0)),
    out_shape=jax.ShapeDtypeStruct((N_PAD, DIM), jnp.float32),
)


def _tc_c_body(aggp_ref, h2_ref, degp_ref, b2_ref, out_ref):
  dis = _dis_from_degp(degp_ref)
  out_ref[...] = (aggp_ref[0] + aggp_ref[1] + h2_ref[...]) * dis[:, None] + \
      b2_ref[...]


_tc_c = pl.pallas_call(
    _tc_c_body,
    grid=(N_PAD // BLK,),
    in_specs=[
        pl.BlockSpec((NC, BLK, DIM), lambda i: (0, i, 0)),
        pl.BlockSpec((BLK, DIM), lambda i: (i, 0)),
        pl.BlockSpec((NC, BLK, DEG_W), lambda i: (0, i, 0)),
        pl.BlockSpec((1, DIM), lambda i: (0, 0)),
    ],
    out_specs=pl.BlockSpec((BLK, DIM), lambda i: (i, 0)),
    out_shape=jax.ShapeDtypeStruct((N_PAD, DIM), jnp.float32),
)


def kernel(x, edge_index, W1, b1, W2, b2):
  xp = jnp.pad(x, ((0, N_PAD - N_NODES), (0, 0)))
  zeros_feat = jnp.zeros((N_PAD, DIM), jnp.float32)
  src1d = edge_index[0].astype(jnp.int32)
  dst1d = edge_index[1].astype(jnp.int32)
  degp = _get_deg_kernel()(dst1d, zeros_feat,
                           jnp.ones((128, DEG_W), jnp.float32))
  h1 = _tc_a(xp, W1, degp)
  aggp1 = _get_agg_kernel()(src1d, dst1d, h1, zeros_feat)
  h2 = _tc_b(aggp1, h1, degp, b1.reshape(1, DIM), W2)
  aggp2 = _get_agg_kernel()(src1d, dst1d, h2, zeros_feat)
  return _tc_c(aggp2, h2, degp, b2.reshape(1, DIM))[:N_NODES]
